# baseline (speedup 1.0000x reference)
import functools

import jax
import jax.numpy as jnp
from jax import lax
from jax.experimental import pallas as pl
from jax.experimental.pallas import tpu as pltpu
from jax.experimental.pallas import tpu_sc as plsc

B = 8
N = 1024
K = 20
NEG = 0.2

_NC = 2
_NS = 16
_NW = _NC * _NS
_PC = 4


def _lrelu(v):
    return jnp.maximum(v, NEG * v)


def _knn_body(x_ref, a1_ref, c1_ref, b1_ref, idx_ref, p1_ref, q1_ref, v_ref):
    for u in range(2):
        xb = x_ref[u]
        inner = lax.dot_general(xb, xb, (((0,), (0,)), ((), ())),
                                preferred_element_type=jnp.float32)
        xx = jnp.sum(xb * xb, axis=0)
        v_ref[u] = 2.0 * inner - xx[:, None] - xx[None, :]

    lane = lax.broadcasted_iota(jnp.int32, (2, N, N), 2)
    kcol = lax.broadcasted_iota(jnp.int32, (2, N, K), 2)

    def step(t, idxacc):
        v = v_ref[...]
        m = jnp.max(v, axis=2, keepdims=True)
        am = jnp.min(jnp.where(v == m, lane, N), axis=2, keepdims=True)
        v_ref[...] = jnp.where(lane == am, -1e30, v)
        return jnp.where(kcol == t, am, idxacc)

    idxacc = lax.fori_loop(0, K, step, jnp.zeros((2, N, K), jnp.int32))
    b = pl.program_id(0)
    boff = 2 * b * N + lax.broadcasted_iota(jnp.int32, (2, N, K), 0) * N
    idx_ref[...] = idxacc + boff

    for u in range(2):
        xb = x_ref[u]
        p1_ref[u] = lax.dot_general(xb, a1_ref[...], (((0,), (1,)), ((), ())),
                                    preferred_element_type=jnp.float32)
        q1_ref[u] = lax.dot_general(xb, c1_ref[...], (((0,), (1,)), ((), ())),
                                    preferred_element_type=jnp.float32) + b1_ref[...][None, :]


def _knn_pq1(x, a1, c1, b1):
    op = a1.shape[0]
    oq = c1.shape[0]
    nb = x.shape[0]
    return pl.pallas_call(
        _knn_body,
        grid=(nb // 2,),
        in_specs=[
            pl.BlockSpec((2, 3, N), lambda b: (b, 0, 0)),
            pl.BlockSpec((op, 3), lambda b: (0, 0)),
            pl.BlockSpec((oq, 3), lambda b: (0, 0)),
            pl.BlockSpec((oq,), lambda b: (0,)),
        ],
        out_specs=[
            pl.BlockSpec((2, N, K), lambda b: (b, 0, 0)),
            pl.BlockSpec((2, N, op), lambda b: (b, 0, 0)),
            pl.BlockSpec((2, N, oq), lambda b: (b, 0, 0)),
        ],
        out_shape=[
            jax.ShapeDtypeStruct((nb, N, K), jnp.int32),
            jax.ShapeDtypeStruct((nb, N, op), jnp.float32),
            jax.ShapeDtypeStruct((nb, N, oq), jnp.float32),
        ],
        scratch_shapes=[pltpu.VMEM((2, N, N), jnp.float32)],
    )(x, a1, c1, b1)


def _gmax_body(o, tw, pts, p_hbm, idx_hbm, out_hbm, idx_v, rows0, rows1, out_v,
               sem0, sem1):
    ch = pts // _PC
    wid = lax.axis_index("s") * _NC + lax.axis_index("c")
    npt = _PC * K
    pltpu.sync_copy(idx_hbm.at[pl.ds(wid * (ch * npt), ch * npt)], idx_v)

    def _idx(c):
        return idx_v.at[pl.ds(pl.multiple_of(c * npt, 8), npt)]

    def start(c, buf, sem):
        pltpu.make_async_copy(p_hbm.at[_idx(c)], buf, sem).start()

    def wait(c, buf, sem):
        pltpu.make_async_copy(p_hbm.at[_idx(c)], buf, sem).wait()

    lanes = 16 if p_hbm.dtype == jnp.float32 else 32

    def compute(c, rows):
        def pbody(p, carry):
            base = p * K
            for ob in range(tw // lanes):
                sl = pl.ds(ob * lanes, lanes)
                acc = rows[base, sl]
                for kk in range(1, K):
                    acc = jnp.maximum(acc, rows[base + kk, sl])
                out_v[c * _PC + p, sl] = acc
            return carry
        lax.fori_loop(0, _PC, pbody, 0)

    start(0, rows0, sem0)
    start(1, rows1, sem1)

    def pair(i, carry):
        c0 = 2 * i
        c1 = 2 * i + 1
        wait(c0, rows0, sem0)
        compute(c0, rows0)

        @pl.when(c0 + 2 < ch)
        def _():
            start(c0 + 2, rows0, sem0)

        wait(c1, rows1, sem1)
        compute(c1, rows1)

        @pl.when(c1 + 2 < ch)
        def _():
            start(c1 + 2, rows1, sem1)

        return carry

    lax.fori_loop(0, ch // 2, pair, 0)
    pltpu.sync_copy(out_v, out_hbm.at[pl.ds(wid * pts, pts)])


def _gather_max(p_rows, idx_flat, tw=None):
    o = p_rows.shape[1]
    tw = o if tw is None else tw
    pts = p_rows.shape[0] // _NW
    dt = p_rows.dtype
    tiled = dt == jnp.float32 and o % 128 == 0
    params = pltpu.CompilerParams(use_tc_tiling_on_sc=tiled)
    kern = functools.partial(
        pl.kernel,
        out_type=jax.ShapeDtypeStruct((p_rows.shape[0], o), dt),
        mesh=plsc.VectorSubcoreMesh(core_axis_name="c", subcore_axis_name="s"),
        compiler_params=params,
        scratch_types=[
            pltpu.VMEM((pts * K,), jnp.int32),
            pltpu.VMEM((_PC * K, o), dt),
            pltpu.VMEM((_PC * K, o), dt),
            pltpu.VMEM((pts, o), dt),
            pltpu.SemaphoreType.DMA,
            pltpu.SemaphoreType.DMA,
        ],
    )(functools.partial(_gmax_body, o, tw, pts))
    return kern(p_rows, idx_flat)


def _layer_body(m_ref, q_ref, a_ref, c_ref, b_ref, x_ref, p_ref, q2_ref):
    ci = q_ref.shape[2]
    xb = _lrelu(m_ref[0][:, :ci] + q_ref[0])
    x_ref[0] = xb
    p_ref[0] = lax.dot_general(xb, a_ref[...], (((1,), (1,)), ((), ())),
                               preferred_element_type=jnp.float32)
    q2_ref[0] = lax.dot_general(xb, c_ref[...], (((1,), (1,)), ((), ())),
                                preferred_element_type=jnp.float32) + b_ref[...][None, :]


def _layer_tc(m, q, a, c, bvec):
    mp = m.shape[2]
    ci = q.shape[2]
    op = a.shape[0]
    oq = c.shape[0]
    nb = m.shape[0]
    return pl.pallas_call(
        _layer_body,
        grid=(nb,),
        in_specs=[
            pl.BlockSpec((1, N, mp), lambda b: (b, 0, 0)),
            pl.BlockSpec((1, N, ci), lambda b: (b, 0, 0)),
            pl.BlockSpec((op, ci), lambda b: (0, 0)),
            pl.BlockSpec((oq, ci), lambda b: (0, 0)),
            pl.BlockSpec((oq,), lambda b: (0,)),
        ],
        out_specs=[
            pl.BlockSpec((1, N, ci), lambda b: (b, 0, 0)),
            pl.BlockSpec((1, N, op), lambda b: (b, 0, 0)),
            pl.BlockSpec((1, N, oq), lambda b: (b, 0, 0)),
        ],
        out_shape=[
            jax.ShapeDtypeStruct((nb, N, ci), jnp.float32),
            jax.ShapeDtypeStruct((nb, N, op), jnp.float32),
            jax.ShapeDtypeStruct((nb, N, oq), jnp.float32),
        ],
    )(m, q, a, c, bvec)


def _attend(xb, wq_ref, wk_ref, wv_ref, heads):
    c = xb.shape[1]
    s = N // 4
    dh = c // heads
    srow = lax.broadcasted_iota(jnp.int32, (s, N), 0)
    scol = lax.broadcasted_iota(jnp.int32, (s, N), 1)
    sm = (scol == 4 * srow).astype(jnp.float32)
    xs = lax.dot_general(sm, xb, (((1,), (0,)), ((), ())),
                         preferred_element_type=jnp.float32)
    qm = lax.dot_general(xb, wq_ref[...], (((1,), (1,)), ((), ())),
                         preferred_element_type=jnp.float32)
    km = lax.dot_general(xs, wk_ref[...], (((1,), (1,)), ((), ())),
                         preferred_element_type=jnp.float32)
    vm = lax.dot_general(xs, wv_ref[...], (((1,), (1,)), ((), ())),
                         preferred_element_type=jnp.float32)
    scale = 1.0 / (dh ** 0.5)
    outs = []
    for h in range(heads):
        lo = h * dh
        qh = qm[:, lo:lo + dh]
        kh = km[:, lo:lo + dh]
        vh = vm[:, lo:lo + dh]
        att = lax.dot_general(qh, kh, (((1,), (1,)), ((), ())),
                              preferred_element_type=jnp.float32) * scale
        att = att - jnp.max(att, axis=1, keepdims=True)
        e = jnp.exp(att)
        e = e / jnp.sum(e, axis=1, keepdims=True)
        outs.append(lax.dot_general(e, vh, (((1,), (0,)), ((), ())),
                                    preferred_element_type=jnp.float32))
    return xb + jnp.concatenate(outs, axis=1)


def _attn3_body(m_ref, q_ref, wq_ref, wk_ref, wv_ref, a_ref, c_ref, b_ref,
                x_ref, p_ref, q2_ref):
    xb = _lrelu(m_ref[0] + q_ref[0])
    xb = _attend(xb, wq_ref, wk_ref, wv_ref, 4)
    x_ref[0] = xb
    p_ref[0] = lax.dot_general(xb, a_ref[...], (((1,), (1,)), ((), ())),
                               preferred_element_type=jnp.float32).astype(p_ref.dtype)
    q2_ref[0] = lax.dot_general(xb, c_ref[...], (((1,), (1,)), ((), ())),
                                preferred_element_type=jnp.float32) + b_ref[...][None, :]


def _attn3_tc(m, q, wq, wk, wv, a, c, bvec):
    ci = m.shape[2]
    o = a.shape[0]
    nb = m.shape[0]
    return pl.pallas_call(
        _attn3_body,
        grid=(nb,),
        in_specs=[
            pl.BlockSpec((1, N, ci), lambda b: (b, 0, 0)),
            pl.BlockSpec((1, N, ci), lambda b: (b, 0, 0)),
            pl.BlockSpec((ci, ci), lambda b: (0, 0)),
            pl.BlockSpec((ci, ci), lambda b: (0, 0)),
            pl.BlockSpec((ci, ci), lambda b: (0, 0)),
            pl.BlockSpec((o, ci), lambda b: (0, 0)),
            pl.BlockSpec((o, ci), lambda b: (0, 0)),
            pl.BlockSpec((o,), lambda b: (0,)),
        ],
        out_specs=[
            pl.BlockSpec((1, N, ci), lambda b: (b, 0, 0)),
            pl.BlockSpec((1, N, o), lambda b: (b, 0, 0)),
            pl.BlockSpec((1, N, o), lambda b: (b, 0, 0)),
        ],
        out_shape=[
            jax.ShapeDtypeStruct((nb, N, ci), jnp.float32),
            jax.ShapeDtypeStruct((nb, N, o), jnp.bfloat16),
            jax.ShapeDtypeStruct((nb, N, o), jnp.float32),
        ],
    )(m, q, wq, wk, wv, a, c, bvec)


def _attn4_body(m_ref, q_ref, wq_ref, wk_ref, wv_ref, x1_ref, x2_ref, x3_ref,
                a5_ref, b5_ref, h_ref):
    xb = _lrelu(m_ref[0].astype(jnp.float32) + q_ref[0])
    xb = _attend(xb, wq_ref, wk_ref, wv_ref, 4)
    xc = jnp.concatenate([x1_ref[0], x2_ref[0], x3_ref[0], xb], axis=1)
    y = lax.dot_general(xc, a5_ref[...], (((1,), (1,)), ((), ())),
                        preferred_element_type=jnp.float32) + b5_ref[...][None, :]
    y = _lrelu(y)
    hmax = jnp.max(y, axis=0)
    hmean = jnp.sum(y, axis=0) * (1.0 / N)
    h_ref[0, 0] = jnp.concatenate([hmax, hmean])


def _attn4_tc(m, q, wq, wk, wv, x1, x2, x3, a5, b5):
    ci = m.shape[2]
    nb = m.shape[0]
    return pl.pallas_call(
        _attn4_body,
        grid=(nb,),
        in_specs=[
            pl.BlockSpec((1, N, ci), lambda b: (b, 0, 0)),
            pl.BlockSpec((1, N, ci), lambda b: (b, 0, 0)),
            pl.BlockSpec((ci, ci), lambda b: (0, 0)),
            pl.BlockSpec((ci, ci), lambda b: (0, 0)),
            pl.BlockSpec((ci, ci), lambda b: (0, 0)),
            pl.BlockSpec((1, N, 64), lambda b: (b, 0, 0)),
            pl.BlockSpec((1, N, 64), lambda b: (b, 0, 0)),
            pl.BlockSpec((1, N, 128), lambda b: (b, 0, 0)),
            pl.BlockSpec((1024, 512), lambda b: (0, 0)),
            pl.BlockSpec((1024,), lambda b: (0,)),
        ],
        out_specs=[pl.BlockSpec((1, 1, 2048), lambda b: (b, 0, 0))],
        out_shape=[jax.ShapeDtypeStruct((nb, 1, 2048), jnp.float32)],
    )(m, q, wq, wk, wv, x1, x2, x3, a5, b5)


def _head_body(h_ref, l1_ref, b18_ref, l2_ref, c2_ref, l3_ref, b3_ref, o_ref):
    t = _lrelu(lax.dot_general(h_ref[...], l1_ref[...], (((1,), (1,)), ((), ())),
                               preferred_element_type=jnp.float32) + b18_ref[...][None, :])
    t = _lrelu(lax.dot_general(t, l2_ref[...], (((1,), (1,)), ((), ())),
                               preferred_element_type=jnp.float32) + c2_ref[...][None, :])
    o_ref[...] = lax.dot_general(t, l3_ref[...], (((1,), (1,)), ((), ())),
                                 preferred_element_type=jnp.float32) + b3_ref[...][None, :]


def _head_tc(h, l1, b18, l2, c2, l3, b3):
    return pl.pallas_call(
        _head_body,
        out_shape=jax.ShapeDtypeStruct((B, 40), jnp.float32),
    )(h, l1, b18, l2, c2, l3, b3)


def kernel(x, W1, g1, b1, W2, g2, b2, W3, g3, b3, W4, g4, b4,
           Wq3, Wk3, Wv3, Wq4, Wk4, Wv4, W5, g5, b5,
           L1, g18, b18, L2, bias2, g19, b19, L3, bias3):
    def split(w, g):
        c = w.shape[1] // 2
        wn, wc = w[:, :c], w[:, c:]
        return g[:, None] * wn, g[:, None] * (wc - wn)

    a1, c1 = split(W1, g1)
    a2, c2_ = split(W2, g2)
    a3, c3 = split(W3, g3)
    a4, c4 = split(W4, g4)
    a5 = g5[:, None] * W5
    l1f = g18[:, None] * L1
    l2f = g19[:, None] * L2
    cb2 = g19 * bias2 + b19

    def half(xh):
        nb = xh.shape[0]
        pts = nb * N
        idx, p1, q1 = _knn_pq1(xh, a1, c1, b1)
        idx_flat = idx.reshape(pts * K)

        m1 = _gather_max(p1.reshape(pts, 64), idx_flat).reshape(nb, N, 64)
        x1, p2, q2 = _layer_tc(m1, q1, a2, c2_, b2)

        m2 = _gather_max(p2.reshape(pts, 64), idx_flat).reshape(nb, N, 64)
        x2, p3, q3 = _layer_tc(m2, q2, a3, c3, b3)

        m3 = _gather_max(p3.reshape(pts, 128), idx_flat).reshape(nb, N, 128)
        x3, p4, q4 = _attn3_tc(m3, q3, Wq3, Wk3, Wv3, a4, c4, b4)

        m4 = _gather_max(p4.reshape(pts, 256), idx_flat).reshape(nb, N, 256)
        (h,) = _attn4_tc(m4, q4, Wq4, Wk4, Wv4, x1, x2, x3, a5, b5)
        return h.reshape(nb, 2048)

    h = jnp.concatenate([half(x[:B // 2]), half(x[B // 2:])], axis=0)
    return _head_tc(h, l1f, b18, l2f, cb2, L3, bias3)


# speedup vs baseline: 1.0086x; 1.0086x over previous
import functools

import jax
import jax.numpy as jnp
from jax import lax
from jax.experimental import pallas as pl
from jax.experimental.pallas import tpu as pltpu
from jax.experimental.pallas import tpu_sc as plsc

B = 8
N = 1024
K = 20
NEG = 0.2

_NC = 2
_NS = 16
_NW = _NC * _NS
_PC = 4


def _lrelu(v):
    return jnp.maximum(v, NEG * v)


def _knn_body(x_ref, a1_ref, c1_ref, b1_ref, idx_ref, p1_ref, q1_ref, v_ref):
    for u in range(2):
        xb = x_ref[u]
        inner = lax.dot_general(xb, xb, (((0,), (0,)), ((), ())),
                                preferred_element_type=jnp.float32)
        xx = jnp.sum(xb * xb, axis=0)
        v_ref[u] = 2.0 * inner - xx[:, None] - xx[None, :]

    lane = lax.broadcasted_iota(jnp.int32, (2, N, N), 2)
    kcol = lax.broadcasted_iota(jnp.int32, (2, N, K), 2)

    def step(t, idxacc):
        v = v_ref[...]
        m = jnp.max(v, axis=2, keepdims=True)
        am = jnp.min(jnp.where(v == m, lane, N), axis=2, keepdims=True)
        v_ref[...] = jnp.where(lane == am, -1e30, v)
        return jnp.where(kcol == t, am, idxacc)

    idxacc = lax.fori_loop(0, K, step, jnp.zeros((2, N, K), jnp.int32))
    b = pl.program_id(0)
    boff = 2 * b * N + lax.broadcasted_iota(jnp.int32, (2, N, K), 0) * N
    idx_ref[...] = idxacc + boff

    for u in range(2):
        xb = x_ref[u]
        p1_ref[u] = lax.dot_general(xb, a1_ref[...], (((0,), (1,)), ((), ())),
                                    preferred_element_type=jnp.float32)
        q1_ref[u] = lax.dot_general(xb, c1_ref[...], (((0,), (1,)), ((), ())),
                                    preferred_element_type=jnp.float32) + b1_ref[...][None, :]


def _knn_pq1(x, a1, c1, b1):
    op = a1.shape[0]
    oq = c1.shape[0]
    nb = x.shape[0]
    return pl.pallas_call(
        _knn_body,
        grid=(nb // 2,),
        in_specs=[
            pl.BlockSpec((2, 3, N), lambda b: (b, 0, 0)),
            pl.BlockSpec((op, 3), lambda b: (0, 0)),
            pl.BlockSpec((oq, 3), lambda b: (0, 0)),
            pl.BlockSpec((oq,), lambda b: (0,)),
        ],
        out_specs=[
            pl.BlockSpec((2, N, K), lambda b: (b, 0, 0)),
            pl.BlockSpec((2, N, op), lambda b: (b, 0, 0)),
            pl.BlockSpec((2, N, oq), lambda b: (b, 0, 0)),
        ],
        out_shape=[
            jax.ShapeDtypeStruct((nb, N, K), jnp.int32),
            jax.ShapeDtypeStruct((nb, N, op), jnp.float32),
            jax.ShapeDtypeStruct((nb, N, oq), jnp.float32),
        ],
        scratch_shapes=[pltpu.VMEM((2, N, N), jnp.float32)],
    )(x, a1, c1, b1)


def _gmax_body(o, tw, pts, p_hbm, idx_hbm, out_hbm, idx_v, rows0, rows1, out_v,
               sem0, sem1):
    ch = pts // _PC
    wid = lax.axis_index("s") * _NC + lax.axis_index("c")
    npt = _PC * K
    pltpu.sync_copy(idx_hbm.at[pl.ds(wid * (ch * npt), ch * npt)], idx_v)

    def _idx(c):
        return idx_v.at[pl.ds(pl.multiple_of(c * npt, 8), npt)]

    def start(c, buf, sem):
        pltpu.make_async_copy(p_hbm.at[_idx(c)], buf, sem).start()

    def wait(c, buf, sem):
        pltpu.make_async_copy(p_hbm.at[_idx(c)], buf, sem).wait()

    lanes = 16 if p_hbm.dtype == jnp.float32 else 32

    def compute(c, rows):
        def pbody(p, carry):
            base = p * K
            for ob in range(tw // lanes):
                sl = pl.ds(ob * lanes, lanes)
                acc = rows[base, sl]
                for kk in range(1, K):
                    acc = jnp.maximum(acc, rows[base + kk, sl])
                out_v[c * _PC + p, sl] = acc
            return carry
        lax.fori_loop(0, _PC, pbody, 0)

    start(0, rows0, sem0)
    start(1, rows1, sem1)

    def pair(i, carry):
        c0 = 2 * i
        c1 = 2 * i + 1
        wait(c0, rows0, sem0)
        compute(c0, rows0)

        @pl.when(c0 + 2 < ch)
        def _():
            start(c0 + 2, rows0, sem0)

        wait(c1, rows1, sem1)
        compute(c1, rows1)

        @pl.when(c1 + 2 < ch)
        def _():
            start(c1 + 2, rows1, sem1)

        return carry

    lax.fori_loop(0, ch // 2, pair, 0)
    pltpu.sync_copy(out_v, out_hbm.at[pl.ds(wid * pts, pts)])


def _gather_max(p_rows, idx_flat, tw=None):
    o = p_rows.shape[1]
    tw = o if tw is None else tw
    pts = p_rows.shape[0] // _NW
    dt = p_rows.dtype
    tiled = dt == jnp.float32 and o % 128 == 0
    params = pltpu.CompilerParams(use_tc_tiling_on_sc=tiled)
    kern = functools.partial(
        pl.kernel,
        out_type=jax.ShapeDtypeStruct((p_rows.shape[0], o), dt),
        mesh=plsc.VectorSubcoreMesh(core_axis_name="c", subcore_axis_name="s"),
        compiler_params=params,
        scratch_types=[
            pltpu.VMEM((pts * K,), jnp.int32),
            pltpu.VMEM((_PC * K, o), dt),
            pltpu.VMEM((_PC * K, o), dt),
            pltpu.VMEM((pts, o), dt),
            pltpu.SemaphoreType.DMA,
            pltpu.SemaphoreType.DMA,
        ],
    )(functools.partial(_gmax_body, o, tw, pts))
    return kern(p_rows, idx_flat)


def _layer_body(m_ref, q_ref, a_ref, c_ref, b_ref, x_ref, p_ref, q2_ref):
    ci = q_ref.shape[2]
    xb = _lrelu(m_ref[0][:, :ci] + q_ref[0])
    x_ref[0] = xb
    p_ref[0] = lax.dot_general(xb, a_ref[...], (((1,), (1,)), ((), ())),
                               preferred_element_type=jnp.float32)
    q2_ref[0] = lax.dot_general(xb, c_ref[...], (((1,), (1,)), ((), ())),
                                preferred_element_type=jnp.float32) + b_ref[...][None, :]


def _layer_tc(m, q, a, c, bvec):
    mp = m.shape[2]
    ci = q.shape[2]
    op = a.shape[0]
    oq = c.shape[0]
    nb = m.shape[0]
    return pl.pallas_call(
        _layer_body,
        grid=(nb,),
        in_specs=[
            pl.BlockSpec((1, N, mp), lambda b: (b, 0, 0)),
            pl.BlockSpec((1, N, ci), lambda b: (b, 0, 0)),
            pl.BlockSpec((op, ci), lambda b: (0, 0)),
            pl.BlockSpec((oq, ci), lambda b: (0, 0)),
            pl.BlockSpec((oq,), lambda b: (0,)),
        ],
        out_specs=[
            pl.BlockSpec((1, N, ci), lambda b: (b, 0, 0)),
            pl.BlockSpec((1, N, op), lambda b: (b, 0, 0)),
            pl.BlockSpec((1, N, oq), lambda b: (b, 0, 0)),
        ],
        out_shape=[
            jax.ShapeDtypeStruct((nb, N, ci), jnp.float32),
            jax.ShapeDtypeStruct((nb, N, op), jnp.float32),
            jax.ShapeDtypeStruct((nb, N, oq), jnp.float32),
        ],
    )(m, q, a, c, bvec)


def _attend(xb, wq_ref, wk_ref, wv_ref, heads):
    c = xb.shape[1]
    s = N // 4
    dh = c // heads
    srow = lax.broadcasted_iota(jnp.int32, (s, N), 0)
    scol = lax.broadcasted_iota(jnp.int32, (s, N), 1)
    sm = (scol == 4 * srow).astype(jnp.float32)
    xs = lax.dot_general(sm, xb, (((1,), (0,)), ((), ())),
                         preferred_element_type=jnp.float32)
    qm = lax.dot_general(xb, wq_ref[...], (((1,), (1,)), ((), ())),
                         preferred_element_type=jnp.float32)
    km = lax.dot_general(xs, wk_ref[...], (((1,), (1,)), ((), ())),
                         preferred_element_type=jnp.float32)
    vm = lax.dot_general(xs, wv_ref[...], (((1,), (1,)), ((), ())),
                         preferred_element_type=jnp.float32)
    scale = 1.0 / (dh ** 0.5)
    outs = []
    for h in range(heads):
        lo = h * dh
        qh = qm[:, lo:lo + dh]
        kh = km[:, lo:lo + dh]
        vh = vm[:, lo:lo + dh]
        att = lax.dot_general(qh, kh, (((1,), (1,)), ((), ())),
                              preferred_element_type=jnp.float32) * scale
        att = att - jnp.max(att, axis=1, keepdims=True)
        e = jnp.exp(att)
        e = e / jnp.sum(e, axis=1, keepdims=True)
        outs.append(lax.dot_general(e, vh, (((1,), (0,)), ((), ())),
                                    preferred_element_type=jnp.float32))
    return xb + jnp.concatenate(outs, axis=1)


def _attn3_body(m_ref, q_ref, wq_ref, wk_ref, wv_ref, a_ref, c_ref, b_ref,
                x_ref, p_ref, q2_ref):
    xb = _lrelu(m_ref[0] + q_ref[0])
    xb = _attend(xb, wq_ref, wk_ref, wv_ref, 4)
    x_ref[0] = xb
    p_ref[0] = lax.dot_general(xb, a_ref[...], (((1,), (1,)), ((), ())),
                               preferred_element_type=jnp.float32).astype(p_ref.dtype)
    q2_ref[0] = lax.dot_general(xb, c_ref[...], (((1,), (1,)), ((), ())),
                                preferred_element_type=jnp.float32) + b_ref[...][None, :]


def _attn3_tc(m, q, wq, wk, wv, a, c, bvec):
    ci = m.shape[2]
    o = a.shape[0]
    nb = m.shape[0]
    return pl.pallas_call(
        _attn3_body,
        grid=(nb,),
        in_specs=[
            pl.BlockSpec((1, N, ci), lambda b: (b, 0, 0)),
            pl.BlockSpec((1, N, ci), lambda b: (b, 0, 0)),
            pl.BlockSpec((ci, ci), lambda b: (0, 0)),
            pl.BlockSpec((ci, ci), lambda b: (0, 0)),
            pl.BlockSpec((ci, ci), lambda b: (0, 0)),
            pl.BlockSpec((o, ci), lambda b: (0, 0)),
            pl.BlockSpec((o, ci), lambda b: (0, 0)),
            pl.BlockSpec((o,), lambda b: (0,)),
        ],
        out_specs=[
            pl.BlockSpec((1, N, ci), lambda b: (b, 0, 0)),
            pl.BlockSpec((1, N, o), lambda b: (b, 0, 0)),
            pl.BlockSpec((1, N, o), lambda b: (b, 0, 0)),
        ],
        out_shape=[
            jax.ShapeDtypeStruct((nb, N, ci), jnp.float32),
            jax.ShapeDtypeStruct((nb, N, o), jnp.float32),
            jax.ShapeDtypeStruct((nb, N, o), jnp.float32),
        ],
    )(m, q, wq, wk, wv, a, c, bvec)


def _attn4_body(m_ref, q_ref, wq_ref, wk_ref, wv_ref, x1_ref, x2_ref, x3_ref,
                a5_ref, b5_ref, h_ref):
    xb = _lrelu(m_ref[0].astype(jnp.float32) + q_ref[0])
    xb = _attend(xb, wq_ref, wk_ref, wv_ref, 4)
    xc = jnp.concatenate([x1_ref[0], x2_ref[0], x3_ref[0], xb], axis=1)
    y = lax.dot_general(xc, a5_ref[...], (((1,), (1,)), ((), ())),
                        preferred_element_type=jnp.float32) + b5_ref[...][None, :]
    y = _lrelu(y)
    hmax = jnp.max(y, axis=0)
    hmean = jnp.sum(y, axis=0) * (1.0 / N)
    h_ref[0, 0] = jnp.concatenate([hmax, hmean])


def _attn4_tc(m, q, wq, wk, wv, x1, x2, x3, a5, b5):
    ci = m.shape[2]
    nb = m.shape[0]
    return pl.pallas_call(
        _attn4_body,
        grid=(nb,),
        in_specs=[
            pl.BlockSpec((1, N, ci), lambda b: (b, 0, 0)),
            pl.BlockSpec((1, N, ci), lambda b: (b, 0, 0)),
            pl.BlockSpec((ci, ci), lambda b: (0, 0)),
            pl.BlockSpec((ci, ci), lambda b: (0, 0)),
            pl.BlockSpec((ci, ci), lambda b: (0, 0)),
            pl.BlockSpec((1, N, 64), lambda b: (b, 0, 0)),
            pl.BlockSpec((1, N, 64), lambda b: (b, 0, 0)),
            pl.BlockSpec((1, N, 128), lambda b: (b, 0, 0)),
            pl.BlockSpec((1024, 512), lambda b: (0, 0)),
            pl.BlockSpec((1024,), lambda b: (0,)),
        ],
        out_specs=[pl.BlockSpec((1, 1, 2048), lambda b: (b, 0, 0))],
        out_shape=[jax.ShapeDtypeStruct((nb, 1, 2048), jnp.float32)],
    )(m, q, wq, wk, wv, x1, x2, x3, a5, b5)


def _head_body(h_ref, l1_ref, b18_ref, l2_ref, c2_ref, l3_ref, b3_ref, o_ref):
    t = _lrelu(lax.dot_general(h_ref[...], l1_ref[...], (((1,), (1,)), ((), ())),
                               preferred_element_type=jnp.float32) + b18_ref[...][None, :])
    t = _lrelu(lax.dot_general(t, l2_ref[...], (((1,), (1,)), ((), ())),
                               preferred_element_type=jnp.float32) + c2_ref[...][None, :])
    o_ref[...] = lax.dot_general(t, l3_ref[...], (((1,), (1,)), ((), ())),
                                 preferred_element_type=jnp.float32) + b3_ref[...][None, :]


def _head_tc(h, l1, b18, l2, c2, l3, b3):
    return pl.pallas_call(
        _head_body,
        out_shape=jax.ShapeDtypeStruct((B, 40), jnp.float32),
    )(h, l1, b18, l2, c2, l3, b3)


def kernel(x, W1, g1, b1, W2, g2, b2, W3, g3, b3, W4, g4, b4,
           Wq3, Wk3, Wv3, Wq4, Wk4, Wv4, W5, g5, b5,
           L1, g18, b18, L2, bias2, g19, b19, L3, bias3):
    def split(w, g):
        c = w.shape[1] // 2
        wn, wc = w[:, :c], w[:, c:]
        return g[:, None] * wn, g[:, None] * (wc - wn)

    a1, c1 = split(W1, g1)
    a2, c2_ = split(W2, g2)
    a3, c3 = split(W3, g3)
    a4, c4 = split(W4, g4)
    a5 = g5[:, None] * W5
    l1f = g18[:, None] * L1
    l2f = g19[:, None] * L2
    cb2 = g19 * bias2 + b19

    nb = B // 2
    pts = nb * N
    halves = [x[:nb], x[nb:]]

    s = [list(_knn_pq1(xh, a1, c1, b1)) for xh in halves]
    idxf = [si[0].reshape(pts * K) for si in s]

    m1 = [_gather_max(s[i][1].reshape(pts, 64), idxf[i]).reshape(nb, N, 64)
          for i in range(2)]
    l2 = [_layer_tc(m1[i], s[i][2], a2, c2_, b2) for i in range(2)]

    m2 = [_gather_max(l2[i][1].reshape(pts, 64), idxf[i]).reshape(nb, N, 64)
          for i in range(2)]
    l3 = [_layer_tc(m2[i], l2[i][2], a3, c3, b3) for i in range(2)]

    m3 = [_gather_max(l3[i][1].reshape(pts, 128), idxf[i]).reshape(nb, N, 128)
          for i in range(2)]
    l4 = [_attn3_tc(m3[i], l3[i][2], Wq3, Wk3, Wv3, a4, c4, b4)
          for i in range(2)]

    m4 = [_gather_max(l4[i][1].reshape(pts, 256), idxf[i]).reshape(nb, N, 256)
          for i in range(2)]
    hs = [_attn4_tc(m4[i], l4[i][2], Wq4, Wk4, Wv4,
                    l2[i][0], l3[i][0], l4[i][0], a5, b5)[0].reshape(nb, 2048)
          for i in range(2)]

    h = jnp.concatenate(hs, axis=0)
    return _head_tc(h, l1f, b18, l2f, cb2, L3, bias3)
